# baseline (speedup 1.0000x reference)
import functools

import jax
import jax.numpy as jnp
from jax import lax
from jax.experimental import pallas as pl
from jax.experimental.pallas import tpu as pltpu
from jax.experimental.pallas import tpu_sc as plsc

_NENTS = 1000000
_DIM = 64
_B = 16384
_TB = 2 * _B
_MARGIN = 1.0
_L2REG = 0.1

_NW = 32
_TPW = _TB // _NW
_G = 16
_NG = _TPW // _G
_DEPTH = 4


def _sc_body(hidx_hbm, ridx_hbm, tidx_hbm, ents_hbm, rtab_hbm, out_hbm,
             hidx, ridx, tidx, rtab, hbuf, tbuf, outbuf, *sems):
    sem_h = sems[:_DEPTH]
    sem_t = sems[_DEPTH:]
    c = lax.axis_index("c")
    s = lax.axis_index("s")
    wid = s * 2 + c
    base = wid * _TPW
    pltpu.sync_copy(hidx_hbm.at[pl.ds(base, _TPW)], hidx)
    pltpu.sync_copy(ridx_hbm.at[pl.ds(base, _TPW)], ridx)
    pltpu.sync_copy(tidx_hbm.at[pl.ds(base, _TPW)], tidx)
    pltpu.sync_copy(rtab_hbm, rtab)

    def _fire(g, u):
        ivh = hidx[pl.ds(g * _G, _G)]
        ivt = tidx[pl.ds(g * _G, _G)]
        for l in range(_G):
            pltpu.async_copy(
                ents_hbm.at[pl.ds(ivh[l], 1)], hbuf.at[u, pl.ds(l, 1)],
                sem_h[u])
            pltpu.async_copy(
                ents_hbm.at[pl.ds(ivt[l], 1)], tbuf.at[u, pl.ds(l, 1)],
                sem_t[u])

    for u in range(_DEPTH):
        _fire(u, u)

    def _quad(jj, carry):
        for u in range(_DEPTH):
            g = _DEPTH * jj + u
            pltpu.make_async_copy(
                ents_hbm.at[pl.ds(0, _G)], hbuf.at[u], sem_h[u]).wait()
            pltpu.make_async_copy(
                ents_hbm.at[pl.ds(0, _G)], tbuf.at[u], sem_t[u]).wait()

            @pl.when(g + _DEPTH < _NG)
            def _():
                _fire(g + _DEPTH, u)

            ivr = ridx[pl.ds(g * _G, _G)]
            for l in range(_G):
                ri = ivr[l]
                row = lax.shift_right_logical(ri, 1)
                col0 = lax.mul(lax.rem(ri, 2), _DIM)
                acc = jnp.zeros((16,), jnp.float32)
                for k in range(_DIM // 16):
                    hv = hbuf[u, l, pl.ds(k * 16, 16)]
                    tv = tbuf[u, l, pl.ds(k * 16, 16)]
                    rv = rtab[row, pl.ds(col0 + k * 16, 16)]
                    d = (hv + rv) - tv
                    acc = acc + d * d
                outbuf[2 * g + (l // 8), pl.ds((l % 8) * 16, 16)] = acc
        return carry

    lax.fori_loop(0, _NG // _DEPTH, _quad, 0)
    pltpu.sync_copy(outbuf, out_hbm.at[pl.ds(wid * 128, 128)])


@functools.cache
def _sc_scores():
    return functools.partial(
        pl.kernel,
        mesh=plsc.VectorSubcoreMesh(core_axis_name="c", subcore_axis_name="s"),
        cost_estimate=pl.CostEstimate(
            flops=8_000_000, bytes_accessed=20_000_000, transcendentals=0),
        out_type=jax.ShapeDtypeStruct((_TB // 8, 128), jnp.float32),
        scratch_types=[
            pltpu.VMEM((_TPW,), jnp.int32),
            pltpu.VMEM((_TPW,), jnp.int32),
            pltpu.VMEM((_TPW,), jnp.int32),
            pltpu.VMEM((500, 128), jnp.float32),
            pltpu.VMEM((_DEPTH, _G, _DIM), jnp.float32),
            pltpu.VMEM((_DEPTH, _G, _DIM), jnp.float32),
            pltpu.VMEM((128, 128), jnp.float32),
        ] + [pltpu.SemaphoreType.DMA] * (2 * _DEPTH),
    )(_sc_body)


_S3N = 4
_S3STEPS = 25
_B3 = _NENTS // 8 // _S3N // _S3STEPS


def _sweep_body(*refs):
    out_ref = refs[-1]

    @pl.when(pl.program_id(0) == 0)
    def _():
        out_ref[0, 0] = 0.0

    ones = jnp.ones((1, _DIM), jnp.float32)
    tot = jnp.float32(0.0)
    for ref in refs[:-1]:
        x = ref[...].reshape(_B3 * 8, _DIM)
        y = x * x
        s2 = lax.dot_general(ones, y, (((1,), (1,)), ((), ())),
                             preferred_element_type=jnp.float32)
        r = jnp.sqrt(jnp.maximum(s2, 1.0)) - 1.0
        tot = tot + jnp.sum(r)
    out_ref[0, 0] += tot


_sweep_call = pl.pallas_call(
    _sweep_body,
    grid=(_S3STEPS,),
    in_specs=[
        pl.BlockSpec((_B3, 8, _DIM), lambda i, k=k: (k * _S3STEPS + i, 0, 0))
        for k in range(_S3N)
    ],
    out_specs=pl.BlockSpec(memory_space=pltpu.SMEM),
    out_shape=jax.ShapeDtypeStruct((1, 1), jnp.float32),
)


def _sweep(ents_w):
    e3v = ents_w.reshape(_NENTS // 8, 8, _DIM)
    return _sweep_call(*([e3v] * _S3N))


_PR = _TB // 8


def _final_body(part_ref, reg_ref, out_ref):
    x = part_ref[...]
    rows = lax.broadcasted_iota(jnp.int32, (128, 8), 0)
    cols = lax.broadcasted_iota(jnp.int32, (128, 8), 1)
    m = (rows // 16 == cols).astype(jnp.float32)
    sc2 = jnp.dot(x, m, preferred_element_type=jnp.float32)
    scores = jnp.sqrt(sc2)
    gold = scores[: _PR // 2]
    corrupt = scores[_PR // 2:]
    rank = jnp.sum(jnp.maximum(_MARGIN + gold - corrupt, 0.0))
    out_ref[0, 0] = rank + _L2REG * reg_ref[0, 0] + _L2REG * jnp.sum(gold)


_final = pl.pallas_call(
    _final_body,
    in_specs=[
        pl.BlockSpec((_PR, 128), lambda: (0, 0)),
        pl.BlockSpec(memory_space=pltpu.SMEM),
    ],
    out_specs=pl.BlockSpec(memory_space=pltpu.SMEM),
    out_shape=jax.ShapeDtypeStruct((1, 1), jnp.float32),
)


def kernel(heads, rels, tails, sources, heads_bad, rels_bad, tails_bad,
           sources_bad, ents_w, rels_w):
    del sources, sources_bad
    hidx = jnp.concatenate([heads, heads_bad]).astype(jnp.int32)
    ridx = jnp.concatenate([rels, rels_bad]).astype(jnp.int32)
    tidx = jnp.concatenate([tails, tails_bad]).astype(jnp.int32)
    rtab = rels_w.reshape(500, 128)
    part = _sc_scores()(hidx, ridx, tidx, ents_w, rtab)
    reg = _sweep(ents_w)
    out = _final(part, reg)
    return out[0, 0]


# speedup vs baseline: 1.8847x; 1.8847x over previous
import functools

import jax
import jax.numpy as jnp
from jax import lax
from jax.experimental import pallas as pl
from jax.experimental.pallas import tpu as pltpu
from jax.experimental.pallas import tpu_sc as plsc

_NENTS = 1000000
_DIM = 64
_B = 16384
_TB = 2 * _B
_MARGIN = 1.0
_L2REG = 0.1

_C = 16384
_QSTEPS = 31
_HA = _QSTEPS * _C // 2 * 0 + _QSTEPS * _C


def _pack_body(a_ref, b_ref, pack_ref, reg_ref):
    i = pl.program_id(0)

    @pl.when(i == 0)
    def _():
        reg_ref[0, 0] = 0.0

    ones = jnp.ones((1, _DIM), jnp.float32)
    r1 = lax.broadcasted_iota(jnp.int32, (_DIM, _DIM), 0)
    c1 = lax.broadcasted_iota(jnp.int32, (_DIM, _DIM), 1)
    eye = (r1 == c1).astype(jnp.float32)
    lane = lax.broadcasted_iota(jnp.int32, (1, _C), 1)
    tot = jnp.float32(0.0)
    for ref, base in ((a_ref, 0), (b_ref, _HA)):
        x = ref[...]
        y = x * x
        s2 = jnp.dot(ones, y, preferred_element_type=jnp.float32)
        r = jnp.sqrt(jnp.maximum(s2, 1.0)) - 1.0
        col = base + i * _C + lane
        r = jnp.where(col < _NENTS, r, 0.0)
        tot = tot + jnp.sum(r)
    reg_ref[0, 0] += tot

    ta = lax.dot_general(a_ref[...], eye, (((0,), (0,)), ((), ())),
                         preferred_element_type=jnp.float32)
    tb = lax.dot_general(b_ref[...], eye, (((0,), (0,)), ((), ())),
                         preferred_element_type=jnp.float32)
    pack_ref[:, pl.ds(0, _DIM)] = ta
    pack_ref[:, pl.ds(_DIM, _DIM)] = tb


_pack_call = pl.pallas_call(
    _pack_body,
    grid=(_QSTEPS,),
    in_specs=[
        pl.BlockSpec((_DIM, _C), lambda i: (0, i)),
        pl.BlockSpec((_DIM, _C), lambda i: (0, _QSTEPS + i)),
    ],
    out_specs=[
        pl.BlockSpec((_C, 128), lambda i: (i, 0)),
        pl.BlockSpec(memory_space=pltpu.SMEM),
    ],
    out_shape=[
        jax.ShapeDtypeStruct((_HA, 128), jnp.float32),
        jax.ShapeDtypeStruct((1, 1), jnp.float32),
    ],
)

_NW = 32
_TPW = _TB // _NW
_CH = 128
_NCH = _TPW // _CH


def _sc_body(hrow_hbm, hoff_hbm, ridx_hbm, trow_hbm, toff_hbm, pack_hbm,
             rtab_hbm, out_hbm,
             hrow, hoff, ridx, trow, toff, rtab, hrows, trows, outbuf,
             sem_h, sem_t):
    c = lax.axis_index("c")
    s = lax.axis_index("s")
    wid = s * 2 + c
    base = wid * _NCH
    pltpu.sync_copy(hrow_hbm.at[pl.ds(base, _NCH)], hrow)
    pltpu.sync_copy(trow_hbm.at[pl.ds(base, _NCH)], trow)
    pltpu.sync_copy(hoff_hbm.at[pl.ds(wid * _TPW, _TPW)], hoff)
    pltpu.sync_copy(toff_hbm.at[pl.ds(wid * _TPW, _TPW)], toff)
    pltpu.sync_copy(ridx_hbm.at[pl.ds(wid * _TPW, _TPW)], ridx)
    pltpu.sync_copy(rtab_hbm, rtab)

    def _chunk(ci, carry):
        pltpu.async_copy(pack_hbm.at[hrow.at[ci]], hrows, sem_h)
        pltpu.async_copy(pack_hbm.at[trow.at[ci]], trows, sem_t)
        pltpu.make_async_copy(
            pack_hbm.at[pl.ds(0, _CH)], hrows, sem_h).wait()
        pltpu.make_async_copy(
            pack_hbm.at[pl.ds(0, _CH)], trows, sem_t).wait()

        def _q(q, carry2):
            ob = ci * _CH + q * 16
            hofv = hoff[pl.ds(ob, 16)]
            tofv = toff[pl.ds(ob, 16)]
            rv = ridx[pl.ds(ob, 16)]
            for l in range(16):
                j = q * 16 + l
                ho = hofv[l]
                to = tofv[l]
                ri = rv[l]
                rrow = lax.shift_right_logical(ri, 1)
                rcol = lax.mul(lax.rem(ri, 2), _DIM)
                acc = jnp.zeros((16,), jnp.float32)
                for k in range(_DIM // 16):
                    hv = hrows[j, pl.ds(ho + k * 16, 16)]
                    tv = trows[j, pl.ds(to + k * 16, 16)]
                    rvv = rtab[rrow, pl.ds(rcol + k * 16, 16)]
                    d = (hv + rvv) - tv
                    acc = acc + d * d
                outbuf[16 * ci + 2 * q + (l // 8),
                       pl.ds((l % 8) * 16, 16)] = acc
            return carry2

        lax.fori_loop(0, _CH // 16, _q, 0)
        return carry

    lax.fori_loop(0, _NCH, _chunk, 0)
    pltpu.sync_copy(outbuf, out_hbm.at[pl.ds(wid * 128, 128)])


@functools.cache
def _sc_scores():
    return functools.partial(
        pl.kernel,
        mesh=plsc.VectorSubcoreMesh(core_axis_name="c", subcore_axis_name="s"),
        cost_estimate=pl.CostEstimate(
            flops=8_000_000, bytes_accessed=40_000_000, transcendentals=0),
        out_type=jax.ShapeDtypeStruct((_TB // 8, 128), jnp.float32),
        scratch_types=[
            pltpu.VMEM((_NCH, _CH), jnp.int32),
            pltpu.VMEM((_TPW,), jnp.int32),
            pltpu.VMEM((_TPW,), jnp.int32),
            pltpu.VMEM((_NCH, _CH), jnp.int32),
            pltpu.VMEM((_TPW,), jnp.int32),
            pltpu.VMEM((500, 128), jnp.float32),
            pltpu.VMEM((_CH, 128), jnp.float32),
            pltpu.VMEM((_CH, 128), jnp.float32),
            pltpu.VMEM((128, 128), jnp.float32),
            pltpu.SemaphoreType.DMA,
            pltpu.SemaphoreType.DMA,
        ],
    )(_sc_body)


_PR = _TB // 8


def _final_body(part_ref, reg_ref, out_ref):
    x = part_ref[...]
    rows = lax.broadcasted_iota(jnp.int32, (128, 8), 0)
    cols = lax.broadcasted_iota(jnp.int32, (128, 8), 1)
    m = (rows // 16 == cols).astype(jnp.float32)
    sc2 = jnp.dot(x, m, preferred_element_type=jnp.float32)
    scores = jnp.sqrt(sc2)
    gold = scores[: _PR // 2]
    corrupt = scores[_PR // 2:]
    rank = jnp.sum(jnp.maximum(_MARGIN + gold - corrupt, 0.0))
    out_ref[0, 0] = rank + _L2REG * reg_ref[0, 0] + _L2REG * jnp.sum(gold)


_final = pl.pallas_call(
    _final_body,
    in_specs=[
        pl.BlockSpec((_PR, 128), lambda: (0, 0)),
        pl.BlockSpec(memory_space=pltpu.SMEM),
    ],
    out_specs=pl.BlockSpec(memory_space=pltpu.SMEM),
    out_shape=jax.ShapeDtypeStruct((1, 1), jnp.float32),
)


def kernel(heads, rels, tails, sources, heads_bad, rels_bad, tails_bad,
           sources_bad, ents_w, rels_w):
    del sources, sources_bad
    hidx = jnp.concatenate([heads, heads_bad]).astype(jnp.int32)
    ridx = jnp.concatenate([rels, rels_bad]).astype(jnp.int32)
    tidx = jnp.concatenate([tails, tails_bad]).astype(jnp.int32)
    hrow = jnp.where(hidx < _HA, hidx, hidx - _HA).reshape(_TB // _CH, _CH)
    hoff = jnp.where(hidx < _HA, 0, _DIM).astype(jnp.int32)
    trow = jnp.where(tidx < _HA, tidx, tidx - _HA).reshape(_TB // _CH, _CH)
    toff = jnp.where(tidx < _HA, 0, _DIM).astype(jnp.int32)
    rtab = rels_w.reshape(500, 128)
    entsT = ents_w.T
    pack, reg = _pack_call(entsT, entsT)
    part = _sc_scores()(hrow, hoff, ridx, trow, toff, pack, rtab)
    out = _final(part, reg)
    return out[0, 0]


# speedup vs baseline: 1.9190x; 1.0182x over previous
import functools

import jax
import jax.numpy as jnp
from jax import lax
from jax.experimental import pallas as pl
from jax.experimental.pallas import tpu as pltpu
from jax.experimental.pallas import tpu_sc as plsc

_NENTS = 1000000
_DIM = 64
_B = 16384
_TB = 2 * _B
_MARGIN = 1.0
_L2REG = 0.1

_C = 16384
_QSTEPS = 31
_HA = _QSTEPS * _C // 2 * 0 + _QSTEPS * _C


def _pack_body(a_ref, b_ref, pack_ref, reg_ref):
    i = pl.program_id(0)

    @pl.when(i == 0)
    def _():
        reg_ref[0, 0] = 0.0

    ones = jnp.ones((1, _DIM), jnp.float32)
    r1 = lax.broadcasted_iota(jnp.int32, (_DIM, _DIM), 0)
    c1 = lax.broadcasted_iota(jnp.int32, (_DIM, _DIM), 1)
    eye = (r1 == c1).astype(jnp.float32)
    lane = lax.broadcasted_iota(jnp.int32, (1, _C), 1)
    tot = jnp.float32(0.0)
    for ref, base in ((a_ref, 0), (b_ref, _HA)):
        x = ref[...]
        y = x * x
        s2 = jnp.dot(ones, y, preferred_element_type=jnp.float32)
        r = jnp.sqrt(jnp.maximum(s2, 1.0)) - 1.0
        col = base + i * _C + lane
        r = jnp.where(col < _NENTS, r, 0.0)
        tot = tot + jnp.sum(r)
    reg_ref[0, 0] += tot

    ta = lax.dot_general(a_ref[...], eye, (((0,), (0,)), ((), ())),
                         preferred_element_type=jnp.float32)
    tb = lax.dot_general(b_ref[...], eye, (((0,), (0,)), ((), ())),
                         preferred_element_type=jnp.float32)
    pack_ref[:, pl.ds(0, _DIM)] = ta
    pack_ref[:, pl.ds(_DIM, _DIM)] = tb


_pack_call = pl.pallas_call(
    _pack_body,
    grid=(_QSTEPS,),
    in_specs=[
        pl.BlockSpec((_DIM, _C), lambda i: (0, i)),
        pl.BlockSpec((_DIM, _C), lambda i: (0, _QSTEPS + i)),
    ],
    out_specs=[
        pl.BlockSpec((_C, 128), lambda i: (i, 0)),
        pl.BlockSpec(memory_space=pltpu.SMEM),
    ],
    out_shape=[
        jax.ShapeDtypeStruct((_HA, 128), jnp.float32),
        jax.ShapeDtypeStruct((1, 1), jnp.float32),
    ],
)

_NW = 32
_TPW = _TB // _NW
_CH = 64
_NCH = _TPW // _CH


def _sc_body(hrow_hbm, hoff_hbm, ridx_hbm, trow_hbm, toff_hbm, pack_hbm,
             rtab_hbm, out_hbm,
             hrow, hoff, ridx, trow, toff, rtab, hrows, trows, outbuf,
             sem_h, sem_t):
    c = lax.axis_index("c")
    s = lax.axis_index("s")
    wid = s * 2 + c
    base = wid * _NCH
    pltpu.sync_copy(hrow_hbm.at[pl.ds(base, _NCH)], hrow)
    pltpu.sync_copy(trow_hbm.at[pl.ds(base, _NCH)], trow)
    pltpu.sync_copy(hoff_hbm.at[pl.ds(wid * _TPW, _TPW)], hoff)
    pltpu.sync_copy(toff_hbm.at[pl.ds(wid * _TPW, _TPW)], toff)
    pltpu.sync_copy(ridx_hbm.at[pl.ds(wid * _TPW, _TPW)], ridx)
    pltpu.sync_copy(rtab_hbm, rtab)

    def _fire(ci, buf):
        pltpu.async_copy(pack_hbm.at[hrow.at[ci]], hrows.at[buf], sem_h)
        pltpu.async_copy(pack_hbm.at[trow.at[ci]], trows.at[buf], sem_t)

    _fire(0, 0)

    def _chunk(ci, carry):
        cb = lax.rem(ci, 2)

        @pl.when(ci < _NCH - 1)
        def _():
            _fire(ci + 1, lax.rem(ci + 1, 2))

        pltpu.make_async_copy(
            pack_hbm.at[pl.ds(0, _CH)], hrows.at[cb], sem_h).wait()
        pltpu.make_async_copy(
            pack_hbm.at[pl.ds(0, _CH)], trows.at[cb], sem_t).wait()

        def _q(q, carry2):
            ob = ci * _CH + q * 16
            hofv = hoff[pl.ds(ob, 16)]
            tofv = toff[pl.ds(ob, 16)]
            rv = ridx[pl.ds(ob, 16)]
            for l in range(16):
                j = q * 16 + l
                ho = hofv[l]
                to = tofv[l]
                ri = rv[l]
                rrow = lax.shift_right_logical(ri, 1)
                rcol = lax.mul(lax.rem(ri, 2), _DIM)
                acc = jnp.zeros((16,), jnp.float32)
                for k in range(_DIM // 16):
                    hv = hrows[cb, j, pl.ds(ho + k * 16, 16)]
                    tv = trows[cb, j, pl.ds(to + k * 16, 16)]
                    rvv = rtab[rrow, pl.ds(rcol + k * 16, 16)]
                    d = (hv + rvv) - tv
                    acc = acc + d * d
                outbuf[8 * ci + 2 * q + (l // 8),
                       pl.ds((l % 8) * 16, 16)] = acc
            return carry2

        lax.fori_loop(0, _CH // 16, _q, 0)
        return carry

    lax.fori_loop(0, _NCH, _chunk, 0)
    pltpu.sync_copy(outbuf, out_hbm.at[pl.ds(wid * 128, 128)])


@functools.cache
def _sc_scores():
    return functools.partial(
        pl.kernel,
        mesh=plsc.VectorSubcoreMesh(core_axis_name="c", subcore_axis_name="s"),
        cost_estimate=pl.CostEstimate(
            flops=8_000_000, bytes_accessed=40_000_000, transcendentals=0),
        out_type=jax.ShapeDtypeStruct((_TB // 8, 128), jnp.float32),
        scratch_types=[
            pltpu.VMEM((_NCH, _CH), jnp.int32),
            pltpu.VMEM((_TPW,), jnp.int32),
            pltpu.VMEM((_TPW,), jnp.int32),
            pltpu.VMEM((_NCH, _CH), jnp.int32),
            pltpu.VMEM((_TPW,), jnp.int32),
            pltpu.VMEM((500, 128), jnp.float32),
            pltpu.VMEM((2, _CH, 128), jnp.float32),
            pltpu.VMEM((2, _CH, 128), jnp.float32),
            pltpu.VMEM((128, 128), jnp.float32),
            pltpu.SemaphoreType.DMA,
            pltpu.SemaphoreType.DMA,
        ],
    )(_sc_body)


_PR = _TB // 8


def _final_body(part_ref, reg_ref, out_ref):
    x = part_ref[...]
    rows = lax.broadcasted_iota(jnp.int32, (128, 8), 0)
    cols = lax.broadcasted_iota(jnp.int32, (128, 8), 1)
    m = (rows // 16 == cols).astype(jnp.float32)
    sc2 = jnp.dot(x, m, preferred_element_type=jnp.float32)
    scores = jnp.sqrt(sc2)
    gold = scores[: _PR // 2]
    corrupt = scores[_PR // 2:]
    rank = jnp.sum(jnp.maximum(_MARGIN + gold - corrupt, 0.0))
    out_ref[0, 0] = rank + _L2REG * reg_ref[0, 0] + _L2REG * jnp.sum(gold)


_final = pl.pallas_call(
    _final_body,
    in_specs=[
        pl.BlockSpec((_PR, 128), lambda: (0, 0)),
        pl.BlockSpec(memory_space=pltpu.SMEM),
    ],
    out_specs=pl.BlockSpec(memory_space=pltpu.SMEM),
    out_shape=jax.ShapeDtypeStruct((1, 1), jnp.float32),
)


def kernel(heads, rels, tails, sources, heads_bad, rels_bad, tails_bad,
           sources_bad, ents_w, rels_w):
    del sources, sources_bad
    hidx = jnp.concatenate([heads, heads_bad]).astype(jnp.int32)
    ridx = jnp.concatenate([rels, rels_bad]).astype(jnp.int32)
    tidx = jnp.concatenate([tails, tails_bad]).astype(jnp.int32)
    hrow = jnp.where(hidx < _HA, hidx, hidx - _HA).reshape(_TB // _CH, _CH)
    hoff = jnp.where(hidx < _HA, 0, _DIM).astype(jnp.int32)
    trow = jnp.where(tidx < _HA, tidx, tidx - _HA).reshape(_TB // _CH, _CH)
    toff = jnp.where(tidx < _HA, 0, _DIM).astype(jnp.int32)
    rtab = rels_w.reshape(500, 128)
    entsT = ents_w.T
    pack, reg = _pack_call(entsT, entsT)
    part = _sc_scores()(hrow, hoff, ridx, trow, toff, pack, rtab)
    out = _final(part, reg)
    return out[0, 0]
